# baseline (speedup 1.0000x reference)
import jax
import jax.numpy as jnp
from jax import lax
from jax.experimental import pallas as pl
from jax.experimental.pallas import tpu as pltpu

_HIDDEN = 2048
_EXPERTS = 64
_NSTREAM = 4
_TM = 512
_M = 16384
_STEPS = _M // (_NSTREAM * _TM)
_SHARD_BLOCKS = _M // _NSTREAM // _TM


def _router_logits_kernel(x0, x1, x2, x3, w_ref, o_ref):
    i = pl.program_id(0)
    w = w_ref[...]
    for s, x in enumerate((x0, x1, x2, x3)):
        res = lax.dot_general(
            x[...],
            w,
            dimension_numbers=(((1,), (1,)), ((), ())),
            preferred_element_type=jnp.float32,
        )
        o_ref[pl.ds((s * _SHARD_BLOCKS + i) * _TM, _TM), :] = res


def kernel(hidden_states, weight):
    hs = hidden_states.reshape(_M, _HIDDEN)

    def make_spec(s):
        return pl.BlockSpec((_TM, _HIDDEN), lambda i, s=s: (s * _SHARD_BLOCKS + i, 0))

    out = pl.pallas_call(
        _router_logits_kernel,
        grid=(_STEPS,),
        in_specs=[make_spec(s) for s in range(_NSTREAM)]
        + [pl.BlockSpec((_EXPERTS, _HIDDEN), lambda i: (0, 0))],
        out_specs=pl.BlockSpec((_M, _EXPERTS), lambda i: (0, 0)),
        out_shape=jax.ShapeDtypeStruct((_M, _EXPERTS), jnp.float32),
    )(hs, hs, hs, hs, weight)
    return out


# speedup vs baseline: 1.0105x; 1.0105x over previous
import jax
import jax.numpy as jnp
from jax import lax
from jax.experimental import pallas as pl
from jax.experimental.pallas import tpu as pltpu

_HIDDEN = 2048
_EXPERTS = 64
_TC = 512
_NBUF = 5


def _outer(x_hbm, w_ref, o_hbm):
    def _inner(x_blk, o_blk):
        o_blk[...] = lax.dot_general(
            x_blk[...],
            w_ref[...],
            dimension_numbers=(((1,), (1,)), ((), ())),
            preferred_element_type=jnp.float32,
        )

    nchunks = x_hbm.shape[0] // _TC
    pltpu.emit_pipeline(
        _inner,
        grid=(nchunks,),
        in_specs=[
            pl.BlockSpec(
                (_TC, _HIDDEN),
                lambda i: (i, 0),
                pipeline_mode=pl.Buffered(buffer_count=_NBUF),
            )
        ],
        out_specs=[pl.BlockSpec((_TC, _EXPERTS), lambda i: (i, 0))],
    )(x_hbm, o_hbm)


def kernel(hidden_states, weight):
    hs = hidden_states.reshape(-1, _HIDDEN)
    m = hs.shape[0]
    out = pl.pallas_call(
        _outer,
        in_specs=[
            pl.BlockSpec(memory_space=pltpu.MemorySpace.HBM),
            pl.BlockSpec(memory_space=pltpu.MemorySpace.VMEM),
        ],
        out_specs=pl.BlockSpec(memory_space=pltpu.MemorySpace.HBM),
        out_shape=jax.ShapeDtypeStruct((m, _EXPERTS), jnp.float32),
    )(hs, weight)
    return out
